# baseline (speedup 1.0000x reference)
import jax
import jax.numpy as jnp
from jax.experimental import pallas as pl
from jax.experimental.pallas import tpu as pltpu

N = 10000
F = 128
NS = 2
RB = 200
NSTEP = N // (NS * RB)
RB2 = 400


def _p1_body(adjA_ref, adjB_ref, x_ref, W1_ref, b1_ref, W2_ref, Wc_ref,
             u_ref, adj16_ref, s1_ref, v_ref):
    i = pl.program_id(0)

    @pl.when(i == 0)
    def _init():
        s1_ref[...] = jnp.dot(x_ref[...], W1_ref[...],
                              preferred_element_type=jnp.float32)
        v_ref[...] = jnp.dot(W2_ref[...], Wc_ref[...],
                             preferred_element_type=jnp.float32)

    for s, a_ref in enumerate((adjA_ref, adjB_ref)):
        a = a_ref[...]
        z = jnp.dot(a, s1_ref[...], preferred_element_type=jnp.float32)
        z = jnp.maximum(z + b1_ref[...], 0.0)
        u_ref[s * RB:(s + 1) * RB, :] = jnp.dot(
            z, v_ref[...], preferred_element_type=jnp.float32)
        adj16_ref[s * RB:(s + 1) * RB, :] = a.astype(jnp.bfloat16)


def _p2_body(adj16_ref, u_ref, c_ref, out_ref):
    out_ref[...] = jnp.dot(adj16_ref[...], u_ref[...],
                           preferred_element_type=jnp.float32) + c_ref[0, 0]


def kernel(adj, x, W1, b1, W2, b2, Wc, bc, We1, be1, We2, be2, Wfc, bfc):
    del We1, be1, We2, be2, Wfc, bfc
    b1_2d = b1.reshape(1, F)

    adj_specs = [
        pl.BlockSpec((RB, N), lambda i, s=s: (NS * i + s, 0))
        for s in range(NS)
    ]
    u, adj16 = pl.pallas_call(
        _p1_body,
        grid=(NSTEP,),
        in_specs=adj_specs + [
            pl.BlockSpec((N, F), lambda i: (0, 0)),
            pl.BlockSpec((F, F), lambda i: (0, 0)),
            pl.BlockSpec((1, F), lambda i: (0, 0)),
            pl.BlockSpec((F, F), lambda i: (0, 0)),
            pl.BlockSpec((F, 1), lambda i: (0, 0)),
        ],
        out_specs=[
            pl.BlockSpec((NS * RB, 1), lambda i: (i, 0)),
            pl.BlockSpec((NS * RB, N), lambda i: (i, 0)),
        ],
        out_shape=[
            jax.ShapeDtypeStruct((N, 1), jnp.float32),
            jax.ShapeDtypeStruct((N, N), jnp.bfloat16),
        ],
        scratch_shapes=[
            pltpu.VMEM((N, F), jnp.float32),
            pltpu.VMEM((F, 1), jnp.float32),
        ],
    )(adj, adj, x, W1, b1_2d, W2, Wc)

    c = (b2.reshape(1, F) @ Wc + bc).reshape(1, 1)
    u16 = u.astype(jnp.bfloat16)

    label = pl.pallas_call(
        _p2_body,
        grid=(N // RB2,),
        in_specs=[
            pl.BlockSpec((RB2, N), lambda i: (i, 0)),
            pl.BlockSpec((N, 1), lambda i: (0, 0)),
            pl.BlockSpec((1, 1), lambda i: (0, 0), memory_space=pltpu.SMEM),
        ],
        out_specs=pl.BlockSpec((RB2, 1), lambda i: (i, 0)),
        out_shape=jax.ShapeDtypeStruct((N, 1), jnp.float32),
    )(adj16, u16, c)
    return label


# speedup vs baseline: 1.1286x; 1.1286x over previous
import jax
import jax.numpy as jnp
from jax.experimental import pallas as pl
from jax.experimental.pallas import tpu as pltpu

N = 10000
F = 128
RB = 200
NBLK = N // RB
NBUF = 4


def _body(adj_ref, x_ref, W1_ref, b1_ref, W2_ref, b2_ref, Wc_ref, bc_ref,
          out_ref, abuf_ref, sem, u_ref, s1_ref, v_ref):
    i = pl.program_id(0)

    def _copy_in(g, slot):
        row = (g % NBLK) * RB
        return pltpu.make_async_copy(
            adj_ref.at[pl.ds(row, RB), :], abuf_ref.at[slot], sem.at[slot])

    @pl.when(i == 0)
    def _prime():
        for b in range(NBUF):
            _copy_in(b, b).start()
        s1_ref[...] = jnp.dot(x_ref[...], W1_ref[...],
                              preferred_element_type=jnp.float32)
        v_ref[...] = jnp.dot(W2_ref[...], Wc_ref[...],
                             preferred_element_type=jnp.float32)

    slot = jax.lax.rem(i, NBUF)
    _copy_in(i, slot).wait()
    blk = jax.lax.rem(i, NBLK)
    a = abuf_ref[slot]

    @pl.when(i < NBLK)
    def _phase1():
        z = jnp.dot(a, s1_ref[...], preferred_element_type=jnp.float32)
        z = jnp.maximum(z + b1_ref[...], 0.0)
        u_ref[pl.ds(blk * RB, RB), :] = jnp.dot(
            z, v_ref[...], preferred_element_type=jnp.float32)

    @pl.when(i >= NBLK)
    def _phase2():
        c = jnp.dot(b2_ref[...], Wc_ref[...],
                    preferred_element_type=jnp.float32) + bc_ref[...]
        out_ref[pl.ds(blk * RB, RB), :] = jnp.dot(
            a, u_ref[...], preferred_element_type=jnp.float32) + c

    @pl.when(i + NBUF < 2 * NBLK)
    def _next():
        _copy_in(i + NBUF, slot).start()


def kernel(adj, x, W1, b1, W2, b2, Wc, bc, We1, be1, We2, be2, Wfc, bfc):
    del We1, be1, We2, be2, Wfc, bfc
    b1_2d = b1.reshape(1, F)
    b2_2d = b2.reshape(1, F)
    bc_2d = bc.reshape(1, 1)

    label = pl.pallas_call(
        _body,
        grid=(2 * NBLK,),
        in_specs=[
            pl.BlockSpec(memory_space=pl.ANY),
            pl.BlockSpec((N, F), lambda i: (0, 0)),
            pl.BlockSpec((F, F), lambda i: (0, 0)),
            pl.BlockSpec((1, F), lambda i: (0, 0)),
            pl.BlockSpec((F, F), lambda i: (0, 0)),
            pl.BlockSpec((1, F), lambda i: (0, 0)),
            pl.BlockSpec((F, 1), lambda i: (0, 0)),
            pl.BlockSpec((1, 1), lambda i: (0, 0)),
        ],
        out_specs=pl.BlockSpec((N, 1), lambda i: (0, 0)),
        out_shape=jax.ShapeDtypeStruct((N, 1), jnp.float32),
        scratch_shapes=[
            pltpu.VMEM((NBUF, RB, N), jnp.float32),
            pltpu.SemaphoreType.DMA((NBUF,)),
            pltpu.VMEM((N, 1), jnp.float32),
            pltpu.VMEM((N, F), jnp.float32),
            pltpu.VMEM((F, 1), jnp.float32),
        ],
    )(adj, x, W1, b1_2d, W2, b2_2d, Wc, bc_2d)
    return label
